# Initial kernel scaffold; baseline (speedup 1.0000x reference)
import jax
import jax.numpy as jnp
from jax.experimental import pallas as pl


def kernel(edge_index, edge_time, num_nodes):
    raise NotImplementedError("write your pallas kernel here")


# speedup vs baseline: 7.3835x; 7.3835x over previous
import functools

import jax
import jax.numpy as jnp
from jax import lax
from jax.experimental import pallas as pl
from jax.experimental.pallas import tpu as pltpu
from jax.experimental.pallas import tpu_sc as plsc

ALPHA = 0.1
NN = 4096
NE = 131072
NWORD = NN * NN // 32
NC, NS, LANES = 2, 16, 16
NWK = NC * NS
EPW = NE // NWK
CH = EPW // 128
WZ = NWORD // NS


def _mesh():
    return plsc.VectorSubcoreMesh(
        core_axis_name="c", subcore_axis_name="s", num_cores=NC, num_subcores=NS
    )


def _decay_body(t_ref, out_ref):
    t = t_ref[...]
    ct = jnp.max(t)
    out_ref[...] = jnp.exp(-ALPHA * (ct - t))


def _decay_call(t2):
    return pl.pallas_call(
        _decay_body,
        out_shape=jax.ShapeDtypeStruct(t2.shape, jnp.float32),
    )(t2)


def _stage2_body(u_hbm, v_hbm, dec_hbm, bm_out, deg_out,
                 u_v, v_v, dec_v, widx_v, bv_v, zi_v, zf_v, bm_sh, deg_sh, sem):
    c = lax.axis_index("c")
    s = lax.axis_index("s")
    wid = s * NC + c

    def _z16(i, _):
        zi_v[pl.ds(i * LANES, LANES)] = jnp.zeros((LANES,), jnp.int32)
        return 0
    lax.fori_loop(0, 4096 // LANES, _z16, 0)

    def _zf16(i, _):
        zf_v[pl.ds(i * LANES, LANES)] = jnp.zeros((LANES,), jnp.float32)
        return 0
    lax.fori_loop(0, 256 // LANES, _zf16, 0)

    for k in range(WZ // 4096):
        pltpu.sync_copy(zi_v, bm_sh.at[pl.ds(s * WZ + k * 4096, 4096)])
    pltpu.sync_copy(zf_v, deg_sh.at[pl.ds(s * 256, 256)])

    pltpu.sync_copy(u_hbm.at[wid], u_v)
    pltpu.sync_copy(v_hbm.at[wid], v_v)
    pltpu.sync_copy(dec_hbm.at[wid], dec_v)

    def _wb(g, _):
        j = g // 8
        kk = (g % 8) * LANES
        u16 = u_v[j, pl.ds(kk, LANES)]
        v16 = v_v[j, pl.ds(kk, LANES)]
        widx_v[j, pl.ds(kk, LANES)] = u16 * 128 + lax.shift_right_logical(v16, 5)
        bv_v[j, pl.ds(kk, LANES)] = lax.shift_left(jnp.ones((LANES,), jnp.int32),
                                                   v16 & 31)
        return 0
    lax.fori_loop(0, EPW // LANES, _wb, 0)

    plsc.subcore_barrier()

    copies = []
    for j in range(CH):
        copies.append(
            pltpu.async_copy(bv_v.at[j], bm_sh.at[widx_v.at[j]], sem, add=True))
        copies.append(
            pltpu.async_copy(dec_v.at[j], deg_sh.at[u_v.at[j]], sem, add=True))
    for cp in copies:
        cp.wait()

    plsc.subcore_barrier()

    pltpu.sync_copy(bm_sh.at[pl.ds(s * WZ, WZ)], bm_out.at[c, pl.ds(s * WZ, WZ)])
    pltpu.sync_copy(deg_sh.at[pl.ds(s * 256, 256)],
                    deg_out.at[c, pl.ds(s * 256, 256)])


def _stage2(u3, v3, dec3):
    fn = pl.kernel(
        _stage2_body,
        out_type=[
            jax.ShapeDtypeStruct((NC, NWORD), jnp.int32),
            jax.ShapeDtypeStruct((NC, NN), jnp.float32),
        ],
        mesh=_mesh(),
        scratch_types=[
            pltpu.VMEM((CH, 128), jnp.int32),
            pltpu.VMEM((CH, 128), jnp.int32),
            pltpu.VMEM((CH, 128), jnp.float32),
            pltpu.VMEM((CH, 128), jnp.int32),
            pltpu.VMEM((CH, 128), jnp.int32),
            pltpu.VMEM((4096,), jnp.int32),
            pltpu.VMEM((256,), jnp.float32),
            pltpu.VMEM_SHARED((NWORD,), jnp.int32),
            pltpu.VMEM_SHARED((NN,), jnp.float32),
            pltpu.SemaphoreType.DMA,
        ],
    )
    return fn(u3, v3, dec3)


def _stage3_body(bmp_ref, degp_ref, bm_ref, pfx_ref, dis_ref):
    bm = bmp_ref[0] + bmp_ref[1]
    bm_ref[...] = bm

    x = lax.bitcast_convert_type(bm, jnp.uint32)
    x = x - ((x >> 1) & jnp.uint32(0x55555555))
    x = (x & jnp.uint32(0x33333333)) + ((x >> 2) & jnp.uint32(0x33333333))
    x = (x + (x >> 4)) & jnp.uint32(0x0F0F0F0F)
    cnt = (x + (x >> 8) + (x >> 16) + (x >> 24)) & jnp.uint32(0x3F)
    cntf = cnt.astype(jnp.float32)

    r = lax.broadcasted_iota(jnp.int32, (128, 128), 0)
    col = lax.broadcasted_iota(jnp.int32, (128, 128), 1)
    tri = (r <= col).astype(jnp.float32)
    incl = jnp.dot(cntf, tri, preferred_element_type=jnp.float32)

    rowtot = incl[:, 127:128]
    acc = rowtot
    sh = 1
    while sh < 4096:
        z = jnp.zeros((sh, 1), jnp.float32)
        acc = acc + jnp.concatenate([z, acc[: 4096 - sh, :]], axis=0)
        sh *= 2
    rowexcl = acc - rowtot

    pfx_ref[...] = (rowexcl + incl - cntf).astype(jnp.int32)

    deg = degp_ref[0] + degp_ref[1]
    dis_ref[...] = jnp.where(deg > 0, lax.rsqrt(deg), 0.0)


def _stage3(bm_parts, deg_parts):
    return pl.pallas_call(
        _stage3_body,
        out_shape=[
            jax.ShapeDtypeStruct((4096, 128), jnp.int32),
            jax.ShapeDtypeStruct((4096, 128), jnp.int32),
            jax.ShapeDtypeStruct((32, 128), jnp.float32),
        ],
    )(bm_parts, deg_parts)


def _stage4_body(u_hbm, v_hbm, dec_hbm, bm_hbm, pfx_hbm, dis_hbm,
                 rows_out, cols_out, w_out,
                 u_v, v_v, dec_v, widx_v, bmw_v, pfx_v, pos_v, w_v,
                 disu_v, disv_v, sem):
    c = lax.axis_index("c")
    s = lax.axis_index("s")
    wid = s * NC + c

    pltpu.sync_copy(u_hbm.at[wid], u_v)
    pltpu.sync_copy(v_hbm.at[wid], v_v)
    pltpu.sync_copy(dec_hbm.at[wid], dec_v)

    def _wi(g, _):
        j = g // 8
        kk = (g % 8) * LANES
        u16 = u_v[j, pl.ds(kk, LANES)]
        v16 = v_v[j, pl.ds(kk, LANES)]
        widx_v[j, pl.ds(kk, LANES)] = u16 * 128 + lax.shift_right_logical(v16, 5)
        return 0
    lax.fori_loop(0, EPW // LANES, _wi, 0)

    copies = []
    for j in range(CH):
        copies.append(pltpu.async_copy(bm_hbm.at[widx_v.at[j]], bmw_v.at[j], sem))
        copies.append(pltpu.async_copy(pfx_hbm.at[widx_v.at[j]], pfx_v.at[j], sem))
        copies.append(pltpu.async_copy(dis_hbm.at[u_v.at[j]], disu_v.at[j], sem))
        copies.append(pltpu.async_copy(dis_hbm.at[v_v.at[j]], disv_v.at[j], sem))
    for cp in copies:
        cp.wait()

    def _rank(g, _):
        j = g // 8
        kk = (g % 8) * LANES
        u16 = u_v[j, pl.ds(kk, LANES)]
        v16 = v_v[j, pl.ds(kk, LANES)]
        d16 = dec_v[j, pl.ds(kk, LANES)]
        bmw = bmw_v[j, pl.ds(kk, LANES)]
        pfx = pfx_v[j, pl.ds(kk, LANES)]
        bv = lax.shift_left(jnp.ones((LANES,), jnp.int32), v16 & 31)
        m = bmw & (bv - 1)
        m = m - (lax.shift_right_logical(m, 1) & 0x55555555)
        m = (m & 0x33333333) + (lax.shift_right_logical(m, 2) & 0x33333333)
        m = (m + lax.shift_right_logical(m, 4)) & 0x0F0F0F0F
        pc = (m + lax.shift_right_logical(m, 8) + lax.shift_right_logical(m, 16)
              + lax.shift_right_logical(m, 24)) & 0x3F
        pos_v[j, pl.ds(kk, LANES)] = pfx + pc
        du = disu_v[j, pl.ds(kk, LANES)]
        dv = disv_v[j, pl.ds(kk, LANES)]
        w_v[j, pl.ds(kk, LANES)] = du * d16 * dv
        return 0
    lax.fori_loop(0, EPW // LANES, _rank, 0)

    copies = []
    for j in range(CH):
        copies.append(pltpu.async_copy(u_v.at[j], rows_out.at[pos_v.at[j]], sem))
        copies.append(pltpu.async_copy(v_v.at[j], cols_out.at[pos_v.at[j]], sem))
        copies.append(pltpu.async_copy(w_v.at[j], w_out.at[pos_v.at[j]], sem))
    for cp in copies:
        cp.wait()


def _stage4(u3, v3, dec3, bm1, pfx1, dis1):
    fn = pl.kernel(
        _stage4_body,
        out_type=[
            jax.ShapeDtypeStruct((NE,), jnp.int32),
            jax.ShapeDtypeStruct((NE,), jnp.int32),
            jax.ShapeDtypeStruct((NE,), jnp.float32),
        ],
        mesh=_mesh(),
        scratch_types=[
            pltpu.VMEM((CH, 128), jnp.int32),
            pltpu.VMEM((CH, 128), jnp.int32),
            pltpu.VMEM((CH, 128), jnp.float32),
            pltpu.VMEM((CH, 128), jnp.int32),
            pltpu.VMEM((CH, 128), jnp.int32),
            pltpu.VMEM((CH, 128), jnp.int32),
            pltpu.VMEM((CH, 128), jnp.int32),
            pltpu.VMEM((CH, 128), jnp.float32),
            pltpu.VMEM((CH, 128), jnp.float32),
            pltpu.VMEM((CH, 128), jnp.float32),
            pltpu.SemaphoreType.DMA,
        ],
    )
    return fn(u3, v3, dec3, bm1, pfx1, dis1)


def kernel(edge_index, edge_time, num_nodes):
    del num_nodes
    u3 = edge_index[0].astype(jnp.int32).reshape(NWK, CH, 128)
    v3 = edge_index[1].astype(jnp.int32).reshape(NWK, CH, 128)
    dec3 = _decay_call(edge_time.reshape(1024, 128)).reshape(NWK, CH, 128)
    bm_parts, deg_parts = _stage2(u3, v3, dec3)
    bm2, pfx2, dis2 = _stage3(
        bm_parts.reshape(NC, 4096, 128), deg_parts.reshape(NC, 32, 128)
    )
    rows, cols, w = _stage4(
        u3, v3, dec3, bm2.reshape(NWORD), pfx2.reshape(NWORD), dis2.reshape(NN)
    )
    return jnp.stack([rows, cols]), w
